# baseline (speedup 1.0000x reference)
import functools

import jax
import jax.numpy as jnp
from jax import lax
from jax.experimental import pallas as pl
from jax.experimental.pallas import tpu as pltpu
from jax.experimental.pallas import tpu_sc as plsc

B, L = 4096, 50
N = B * L
SKU_DIM, HID, ITEM_DIM = 64, 128, 128

NW = 32
CHUNK = 64
PER_W = N // NW
NCHUNK = PER_W // CHUNK
NSLOT = 4
NSLOT3 = 2

_EPS = 1e-5


def _ln(x, g, b):
    mu = jnp.mean(x, axis=-1, keepdims=True)
    var = jnp.mean((x - mu) ** 2, axis=-1, keepdims=True)
    return (x - mu) * lax.rsqrt(var + _EPS) * g + b


_WBLK = 2000


def _prep_body(cat_t, cat_g, cat_b, price_t, price_g, price_b,
               word_t, fc1_w, fc1_b, c2, p2, w2):
    w2[...] = jnp.dot(word_t[...], fc1_w[384:512, :],
                      preferred_element_type=jnp.float32)

    @pl.when(pl.program_id(0) == 0)
    def _():
        c2[...] = jnp.dot(_ln(cat_t[...], cat_g[...], cat_b[...]),
                          fc1_w[128:256, :],
                          preferred_element_type=jnp.float32) + fc1_b[...]
        p2[...] = jnp.dot(_ln(price_t[...], price_g[...], price_b[...]),
                          fc1_w[256:384, :],
                          preferred_element_type=jnp.float32)


def _prep(cat_t, cat_g, cat_b, price_t, price_g, price_b, word_t, fc1_w, fc1_b):
    n_cat, n_price, n_word = cat_t.shape[0], price_t.shape[0], word_t.shape[0]
    grid = n_word // _WBLK
    full = lambda shape: pl.BlockSpec(shape, lambda i: (0, 0))
    return pl.pallas_call(
        _prep_body,
        grid=(grid,),
        in_specs=[
            full((n_cat, HID)), full((1, HID)), full((1, HID)),
            full((n_price, HID)), full((1, HID)), full((1, HID)),
            pl.BlockSpec((_WBLK, HID), lambda i: (i, 0)),
            full((3 * HID + ITEM_DIM, ITEM_DIM)), full((1, ITEM_DIM)),
        ],
        out_specs=[
            full((n_cat, ITEM_DIM)), full((n_price, ITEM_DIM)),
            pl.BlockSpec((_WBLK, ITEM_DIM), lambda i: (i, 0)),
        ],
        out_shape=[
            jax.ShapeDtypeStruct((n_cat, ITEM_DIM), jnp.float32),
            jax.ShapeDtypeStruct((n_price, ITEM_DIM), jnp.float32),
            jax.ShapeDtypeStruct((n_word, ITEM_DIM), jnp.float32),
        ],
    )(cat_t, cat_g.reshape(1, HID), cat_b.reshape(1, HID),
      price_t, price_g.reshape(1, HID), price_b.reshape(1, HID),
      word_t, fc1_w, fc1_b.reshape(1, ITEM_DIM))


def _pipelined_gathers(tables_idx_bufs_outs, base, gsems, wsems, nslot):
    def do_slot_gather(c, b):
        descs = []
        for table, idx, bufs, _ in tables_idx_bufs_outs:
            descs.append(
                pltpu.async_copy(table.at[idx.at[c]], bufs.at[b], gsems.at[b]))
        return descs

    def outer(g, carry):
        c0 = g * nslot
        gd = []
        for b in range(nslot):
            gd.append(do_slot_gather(c0 + b, b))
        wd = []
        for b in range(nslot):
            for d in gd[b]:
                d.wait()
            off = base + (c0 + b) * CHUNK
            for _, _, bufs, out in tables_idx_bufs_outs:
                wd.append(
                    pltpu.async_copy(bufs.at[b], out.at[pl.ds(off, CHUNK)],
                                     wsems.at[b]))
        for d in wd:
            d.wait()
        return carry

    lax.fori_loop(0, NCHUNK // nslot, outer, 0)


def _gather3_body(cat_idx, price_idx, word_idx, c2, p2, w2,
                  cat_out, price_out, word_out,
                  idx_c, idx_p, idx_w, bufc, bufp, bufw, gsems, wsems):
    wid = lax.axis_index("s") * 2 + lax.axis_index("c")
    base = wid * PER_W
    pltpu.sync_copy(cat_idx.at[wid], idx_c)
    pltpu.sync_copy(price_idx.at[wid], idx_p)
    pltpu.sync_copy(word_idx.at[wid], idx_w)
    _pipelined_gathers(
        [(c2, idx_c, bufc, cat_out), (p2, idx_p, bufp, price_out),
         (w2, idx_w, bufw, word_out)], base, gsems, wsems, NSLOT3)


def _gather3(cat_idx, price_idx, word_idx, c2, p2, w2):
    mesh = plsc.VectorSubcoreMesh(core_axis_name="c", subcore_axis_name="s")
    f = functools.partial(
        pl.kernel,
        mesh=mesh,
        out_type=[
            jax.ShapeDtypeStruct((N, ITEM_DIM), jnp.float32),
            jax.ShapeDtypeStruct((N, ITEM_DIM), jnp.float32),
            jax.ShapeDtypeStruct((N, ITEM_DIM), jnp.float32),
        ],
        scratch_types=[
            pltpu.VMEM((NCHUNK, CHUNK), jnp.int32),
            pltpu.VMEM((NCHUNK, CHUNK), jnp.int32),
            pltpu.VMEM((NCHUNK, CHUNK), jnp.int32),
            pltpu.VMEM((NSLOT3, CHUNK, ITEM_DIM), jnp.float32),
            pltpu.VMEM((NSLOT3, CHUNK, ITEM_DIM), jnp.float32),
            pltpu.VMEM((NSLOT3, CHUNK, ITEM_DIM), jnp.float32),
            pltpu.SemaphoreType.DMA((NSLOT3,)),
            pltpu.SemaphoreType.DMA((NSLOT3,)),
        ],
    )(_gather3_body)
    return f(cat_idx, price_idx, word_idx, c2, p2, w2)


def _gather_sku_body(sku_idx, sku_t, sku_out, idx_s, bufs, gsems, wsems):
    wid = lax.axis_index("s") * 2 + lax.axis_index("c")
    base = wid * PER_W
    pltpu.sync_copy(sku_idx.at[wid], idx_s)
    _pipelined_gathers([(sku_t, idx_s, bufs, sku_out)], base, gsems, wsems, NSLOT)


def _gather_sku(sku_idx, sku_t):
    mesh = plsc.VectorSubcoreMesh(core_axis_name="c", subcore_axis_name="s")
    f = functools.partial(
        pl.kernel,
        mesh=mesh,
        compiler_params=pltpu.CompilerParams(use_tc_tiling_on_sc=False),
        out_type=jax.ShapeDtypeStruct((N, SKU_DIM), jnp.float32),
        scratch_types=[
            pltpu.VMEM((NCHUNK, CHUNK), jnp.int32),
            pltpu.VMEM((NSLOT, CHUNK, SKU_DIM), jnp.float32),
            pltpu.SemaphoreType.DMA((NSLOT,)),
            pltpu.SemaphoreType.DMA((NSLOT,)),
        ],
    )(_gather_sku_body)
    return f(sku_idx, sku_t)


_RBLK = 2048


def _combine_body(sku_rows, c2r, p2r, w2r,
                  sku_g, sku_b, proj_w, proj_b, proj_g, proj_b2, w_s, out):
    x = _ln(sku_rows[...], sku_g[...], sku_b[...])
    x = jnp.dot(x, proj_w[...], preferred_element_type=jnp.float32) + proj_b[...]
    x = jax.nn.relu(_ln(x, proj_g[...], proj_b2[...]))
    x = jnp.dot(x, w_s[...], preferred_element_type=jnp.float32)
    out[...] = jax.nn.relu(x + c2r[...] + p2r[...] + w2r[...])


def _combine(sku_rows, c2r, p2r, w2r, sku_g, sku_b,
             proj_w, proj_b, proj_g, proj_b2, w_s):
    grid = N // _RBLK
    row = lambda d: pl.BlockSpec((_RBLK, d), lambda i: (i, 0))
    full = lambda shape: pl.BlockSpec(shape, lambda i: (0, 0))
    return pl.pallas_call(
        _combine_body,
        grid=(grid,),
        in_specs=[
            row(SKU_DIM), row(ITEM_DIM), row(ITEM_DIM), row(ITEM_DIM),
            full((1, SKU_DIM)), full((1, SKU_DIM)),
            full((SKU_DIM, HID)), full((1, HID)), full((1, HID)), full((1, HID)),
            full((HID, ITEM_DIM)),
        ],
        out_specs=row(ITEM_DIM),
        out_shape=jax.ShapeDtypeStruct((N, ITEM_DIM), jnp.float32),
    )(sku_rows, c2r, p2r, w2r,
      sku_g.reshape(1, SKU_DIM), sku_b.reshape(1, SKU_DIM),
      proj_w, proj_b.reshape(1, HID), proj_g.reshape(1, HID),
      proj_b2.reshape(1, HID), w_s)


def kernel(sku_id, cat_id, price_id, word_ids, sku_table, sku_ln_g, sku_ln_b,
           proj_W, proj_b, proj_ln_g, proj_ln_b, cat_table, cat_ln_g,
           cat_ln_b, price_table, price_ln_g, price_ln_b, word_table,
           fc1_W, fc1_b):
    c2, p2, w2 = _prep(cat_table, cat_ln_g, cat_ln_b,
                       price_table, price_ln_g, price_ln_b,
                       word_table, fc1_W, fc1_b)
    shape_ids = lambda a: a.reshape(NW, NCHUNK, CHUNK).astype(jnp.int32)
    sku_rows = _gather_sku(shape_ids(sku_id), sku_table)
    c2r, p2r, w2r = _gather3(shape_ids(cat_id), shape_ids(price_id),
                             shape_ids(word_ids), c2, p2, w2)
    out = _combine(sku_rows, c2r, p2r, w2r, sku_ln_g, sku_ln_b,
                   proj_W, proj_b, proj_ln_g, proj_ln_b, fc1_W[:HID, :])
    return out.reshape(B, L, ITEM_DIM)


# speedup vs baseline: 1.1618x; 1.1618x over previous
import functools

import jax
import jax.numpy as jnp
from jax import lax
from jax.experimental import pallas as pl
from jax.experimental.pallas import tpu as pltpu
from jax.experimental.pallas import tpu_sc as plsc

B, L = 4096, 50
N = B * L
SKU_DIM, HID, ITEM_DIM = 64, 128, 128
NPRICE = 100

NW = 32
CHUNK = 128
PER_W = N // NW
NCHUNK = PER_W // CHUNK
NSLOT = 2

_EPS = 1e-5


def _ln(x, g, b):
    mu = jnp.mean(x, axis=-1, keepdims=True)
    var = jnp.mean((x - mu) ** 2, axis=-1, keepdims=True)
    return (x - mu) * lax.rsqrt(var + _EPS) * g + b


_WBLK = 4000
_CBLK = 40


def _prep_body(cat_t, cat_g, cat_b, price_t, price_g, price_b,
               word_t, fc1_w, fc1_b, cp2, w2):
    p2 = jnp.dot(_ln(price_t[...], price_g[...], price_b[...]),
                 fc1_w[256:384, :], preferred_element_type=jnp.float32)
    c2 = jnp.dot(_ln(cat_t[...], cat_g[...], cat_b[...]),
                 fc1_w[128:256, :],
                 preferred_element_type=jnp.float32) + fc1_b[...]
    cp2[...] = (c2[:, None, :] + p2[None, :, :]).reshape(_WBLK, ITEM_DIM)
    w2[...] = jnp.dot(word_t[...], fc1_w[384:512, :],
                      preferred_element_type=jnp.float32)


def _prep(cat_t, cat_g, cat_b, price_t, price_g, price_b, word_t, fc1_w, fc1_b):
    n_cat, n_price, n_word = cat_t.shape[0], price_t.shape[0], word_t.shape[0]
    grid = n_word // _WBLK
    full = lambda shape: pl.BlockSpec(shape, lambda i: (0, 0))
    return pl.pallas_call(
        _prep_body,
        grid=(grid,),
        in_specs=[
            pl.BlockSpec((_CBLK, HID), lambda i: (i, 0)),
            full((1, HID)), full((1, HID)),
            full((n_price, HID)), full((1, HID)), full((1, HID)),
            pl.BlockSpec((_WBLK, HID), lambda i: (i, 0)),
            full((3 * HID + ITEM_DIM, ITEM_DIM)), full((1, ITEM_DIM)),
        ],
        out_specs=[
            pl.BlockSpec((_WBLK, ITEM_DIM), lambda i: (i, 0)),
            pl.BlockSpec((_WBLK, ITEM_DIM), lambda i: (i, 0)),
        ],
        out_shape=[
            jax.ShapeDtypeStruct((n_cat * n_price, ITEM_DIM), jnp.float32),
            jax.ShapeDtypeStruct((n_word, ITEM_DIM), jnp.float32),
        ],
    )(cat_t, cat_g.reshape(1, HID), cat_b.reshape(1, HID),
      price_t, price_g.reshape(1, HID), price_b.reshape(1, HID),
      word_t, fc1_w, fc1_b.reshape(1, ITEM_DIM))


def _gather_body(sku_idx, cat_idx, price_idx, word_idx, sku2, cp2, w2,
                 sku_out, cp_out, w_out,
                 idx_s, idx_c, idx_w, bufs, bufc, bufw, gsems, wsem):
    wid = lax.axis_index("s") * 2 + lax.axis_index("c")
    base = wid * PER_W
    pltpu.sync_copy(cat_idx.at[wid], idx_c)
    pltpu.sync_copy(price_idx.at[wid], idx_s)
    pltpu.sync_copy(word_idx.at[wid], idx_w)

    def xform_cp(k, carry):
        sl = pl.ds(k * 16, 16)
        idx_c[sl] = idx_c[sl] * NPRICE + idx_s[sl]
        return carry

    lax.fori_loop(0, PER_W // 16, xform_cp, 0)
    pltpu.sync_copy(sku_idx.at[wid], idx_s)

    def xform_sku(k, carry):
        sl = pl.ds(k * 16, 16)
        idx_s[sl] = lax.shift_right_logical(idx_s[sl], 1)
        return carry

    lax.fori_loop(0, PER_W // 16, xform_sku, 0)

    tabs = [(sku2, idx_s, bufs, sku_out), (cp2, idx_c, bufc, cp_out),
            (w2, idx_w, bufw, w_out)]

    def outer(g, carry):
        c0 = g * NSLOT
        gd = []
        for b in range(NSLOT):
            ds = []
            for table, idx, buf, _ in tabs:
                ds.append(pltpu.async_copy(
                    table.at[idx.at[pl.ds((c0 + b) * CHUNK, CHUNK)]],
                    buf.at[pl.ds(b * CHUNK, CHUNK)], gsems.at[b]))
            gd.append(ds)
        for b in range(NSLOT):
            for d in gd[b]:
                d.wait()
        wd = []
        off = base + c0 * CHUNK
        for _, _, buf, out in tabs:
            wd.append(pltpu.async_copy(
                buf, out.at[pl.ds(off, NSLOT * CHUNK)], wsem))
        for d in wd:
            d.wait()
        return carry

    lax.fori_loop(0, NCHUNK // NSLOT, outer, 0)


def _gather(sku_idx, cat_idx, price_idx, word_idx, sku2, cp2, w2):
    mesh = plsc.VectorSubcoreMesh(core_axis_name="c", subcore_axis_name="s")
    f = functools.partial(
        pl.kernel,
        mesh=mesh,
        out_type=[
            jax.ShapeDtypeStruct((N, ITEM_DIM), jnp.float32),
            jax.ShapeDtypeStruct((N, ITEM_DIM), jnp.float32),
            jax.ShapeDtypeStruct((N, ITEM_DIM), jnp.float32),
        ],
        scratch_types=[
            pltpu.VMEM((PER_W,), jnp.int32),
            pltpu.VMEM((PER_W,), jnp.int32),
            pltpu.VMEM((PER_W,), jnp.int32),
            pltpu.VMEM((NSLOT * CHUNK, ITEM_DIM), jnp.float32),
            pltpu.VMEM((NSLOT * CHUNK, ITEM_DIM), jnp.float32),
            pltpu.VMEM((NSLOT * CHUNK, ITEM_DIM), jnp.float32),
            pltpu.SemaphoreType.DMA((NSLOT,)),
            pltpu.SemaphoreType.DMA,
        ],
    )(_gather_body)
    return f(sku_idx, cat_idx, price_idx, word_idx, sku2, cp2, w2)


_RBLK = 2048


def _combine_body(skup, sid, cpr, w2r,
                  sku_g, sku_b, proj_w, proj_b, proj_g, proj_b2, w_s, out):
    pair = skup[...]
    odd = lax.rem(sid[...], 2) == 1
    x = jnp.where(odd, pair[:, SKU_DIM:], pair[:, :SKU_DIM])
    x = _ln(x, sku_g[...], sku_b[...])
    x = jnp.dot(x, proj_w[...], preferred_element_type=jnp.float32) + proj_b[...]
    x = jax.nn.relu(_ln(x, proj_g[...], proj_b2[...]))
    x = jnp.dot(x, w_s[...], preferred_element_type=jnp.float32)
    out[...] = jax.nn.relu(x + cpr[...] + w2r[...])


def _combine(skup, sid, cpr, w2r, sku_g, sku_b,
             proj_w, proj_b, proj_g, proj_b2, w_s):
    grid = N // _RBLK
    row = lambda d: pl.BlockSpec((_RBLK, d), lambda i: (i, 0))
    full = lambda shape: pl.BlockSpec(shape, lambda i: (0, 0))
    return pl.pallas_call(
        _combine_body,
        grid=(grid,),
        in_specs=[
            row(ITEM_DIM), row(1),
            row(ITEM_DIM), row(ITEM_DIM),
            full((1, SKU_DIM)), full((1, SKU_DIM)),
            full((SKU_DIM, HID)), full((1, HID)), full((1, HID)), full((1, HID)),
            full((HID, ITEM_DIM)),
        ],
        out_specs=row(ITEM_DIM),
        out_shape=jax.ShapeDtypeStruct((N, ITEM_DIM), jnp.float32),
    )(skup, sid, cpr, w2r,
      sku_g.reshape(1, SKU_DIM), sku_b.reshape(1, SKU_DIM),
      proj_w, proj_b.reshape(1, HID), proj_g.reshape(1, HID),
      proj_b2.reshape(1, HID), w_s)


def kernel(sku_id, cat_id, price_id, word_ids, sku_table, sku_ln_g, sku_ln_b,
           proj_W, proj_b, proj_ln_g, proj_ln_b, cat_table, cat_ln_g,
           cat_ln_b, price_table, price_ln_g, price_ln_b, word_table,
           fc1_W, fc1_b):
    cp2, w2 = _prep(cat_table, cat_ln_g, cat_ln_b,
                    price_table, price_ln_g, price_ln_b,
                    word_table, fc1_W, fc1_b)
    sku2 = sku_table.reshape(sku_table.shape[0] // 2, 2 * SKU_DIM)
    shape_ids = lambda a: a.reshape(NW, PER_W).astype(jnp.int32)
    skup, cpr, w2r = _gather(
        shape_ids(sku_id), shape_ids(cat_id), shape_ids(price_id),
        shape_ids(word_ids), sku2, cp2, w2)
    sid = sku_id.reshape(N, 1).astype(jnp.int32)
    out = _combine(skup, sid, cpr, w2r, sku_ln_g, sku_ln_b,
                   proj_W, proj_b, proj_ln_g, proj_ln_b, fc1_W[:HID, :])
    return out.reshape(B, L, ITEM_DIM)
